# baseline (speedup 1.0000x reference)
import functools

import jax
import jax.numpy as jnp
from jax import lax
from jax.experimental import pallas as pl
from jax.experimental.pallas import tpu as pltpu
from jax.experimental.pallas import tpu_sc as plsc

_NC = 2
_NS = 16
_NW = _NC * _NS

_BATCH = 16384
_HEADS = 26
_DIM = 32
_N = _BATCH * _HEADS
_PER_W = _N // _NW
_CH = 832
_NCHUNK = _PER_W // _CH
_LANES = 16


def _body(hash_hbm, table_hbm, off_hbm, out_hbm,
          idx_v, off_v, rows_v, gsem0, gsem1, wsem0, wsem1):
    wid = lax.axis_index("s") * _NC + lax.axis_index("c")
    base = wid * _PER_W

    pltpu.sync_copy(hash_hbm.at[pl.ds(base, _PER_W)], idx_v)
    pltpu.sync_copy(off_hbm, off_v)

    def _add(i, carry):
        o = i * _LANES
        idx_v[pl.ds(o, _LANES)] = idx_v[pl.ds(o, _LANES)] + off_v[pl.ds(o, _LANES)]
        return carry

    lax.fori_loop(0, _PER_W // _LANES, _add, 0)

    gsems = (gsem0, gsem1)
    wsems = (wsem0, wsem1)

    def g_copy(c, s):
        return pltpu.make_async_copy(
            table_hbm.at[idx_v.at[pl.ds(c * _CH, _CH)]], rows_v.at[s], gsems[s])

    def w_copy(c, s):
        return pltpu.make_async_copy(
            rows_v.at[s], out_hbm.at[pl.ds(base + c * _CH, _CH)], wsems[s])

    g_copy(0, 0).start()
    for c in range(_NCHUNK):
        s = c & 1
        g_copy(c, s).wait()
        if c + 1 < _NCHUNK:
            s2 = (c + 1) & 1
            if c + 1 >= 2:
                w_copy(c - 1, s2).wait()
            g_copy(c + 1, s2).start()
        w_copy(c, s).start()
    w_copy(_NCHUNK - 2, (_NCHUNK - 2) & 1).wait()
    w_copy(_NCHUNK - 1, (_NCHUNK - 1) & 1).wait()


@functools.partial(jax.jit, static_argnames=())
def _gather(hash_flat, table, off_tiled):
    mesh = plsc.VectorSubcoreMesh(core_axis_name="c", subcore_axis_name="s")
    k = functools.partial(
        pl.kernel,
        mesh=mesh,
        out_type=jax.ShapeDtypeStruct((_N, _DIM), jnp.float32),
        scratch_types=[
            pltpu.VMEM((_PER_W,), jnp.int32),
            pltpu.VMEM((_PER_W,), jnp.int32),
            pltpu.VMEM((2, _CH, _DIM), jnp.float32),
            pltpu.SemaphoreType.DMA,
            pltpu.SemaphoreType.DMA,
            pltpu.SemaphoreType.DMA,
            pltpu.SemaphoreType.DMA,
        ],
        compiler_params=pltpu.CompilerParams(use_tc_tiling_on_sc=False),
    )(_body)
    return k(hash_flat, table, off_tiled)


def kernel(hash_ids, table, offsets):
    hash_flat = hash_ids.reshape(-1)
    off_tiled = jnp.tile(offsets, _PER_W // _HEADS)
    out = _gather(hash_flat, table, off_tiled)
    return out.reshape(_BATCH, _HEADS, _DIM)
